# baseline (speedup 1.0000x reference)
import jax
import jax.numpy as jnp
from jax.experimental import pallas as pl
from jax.experimental.pallas import tpu as pltpu

IN_F = 4096
OUT_F = 4096
RANK = 64
TOPK = 8
SCALE = 2.0

BT = 256
NJ = 4
FJ = OUT_F // NJ


def _fused_body(x_ref, at_ref, d_ref, bt_ref, o_ref, my_ref):
    @pl.when(pl.program_id(1) == 0)
    def _route():
        y = jnp.dot(x_ref[...].astype(jnp.bfloat16), at_ref[...],
                    preferred_element_type=jnp.float32)
        a = jnp.abs(y + d_ref[...])

        work = a.T
        iota = jax.lax.broadcasted_iota(jnp.int32, (RANK, BT), 0)
        keep = jnp.zeros((RANK, BT), jnp.float32)
        for _ in range(TOPK):
            m = jnp.max(work, axis=0, keepdims=True)
            first = jnp.min(jnp.where(work == m, iota, RANK),
                            axis=0, keepdims=True)
            sel = iota == first
            keep = jnp.where(sel, 1.0, keep)
            work = jnp.where(sel, -1.0, work)
        my_ref[...] = (y * keep.T).astype(jnp.bfloat16)

    out = jnp.dot(my_ref[...], bt_ref[...],
                  preferred_element_type=jnp.float32)
    o_ref[...] = out * SCALE


def kernel(x, A, d, B):
    orig_shape = x.shape
    xt = x.reshape(-1, IN_F)
    n_tok = xt.shape[0]
    grid = (n_tok // BT, NJ)

    out = pl.pallas_call(
        _fused_body,
        grid=grid,
        in_specs=[
            pl.BlockSpec((BT, IN_F), lambda i, j: (i, 0)),
            pl.BlockSpec((IN_F, RANK), lambda i, j: (0, 0)),
            pl.BlockSpec((1, RANK), lambda i, j: (0, 0)),
            pl.BlockSpec((RANK, FJ), lambda i, j: (0, j)),
        ],
        out_specs=pl.BlockSpec((BT, FJ), lambda i, j: (i, j)),
        out_shape=jax.ShapeDtypeStruct((n_tok, OUT_F), jnp.float32),
        scratch_shapes=[pltpu.VMEM((BT, RANK), jnp.bfloat16)],
        compiler_params=pltpu.CompilerParams(
            dimension_semantics=("parallel", "arbitrary")),
    )(xt, A.T.astype(jnp.bfloat16), d.reshape(1, RANK),
      B.T.astype(jnp.bfloat16))

    return out.reshape(orig_shape[:-1] + (OUT_F,))


# speedup vs baseline: 1.9794x; 1.9794x over previous
import jax
import jax.numpy as jnp
from jax.experimental import pallas as pl
from jax.experimental.pallas import tpu as pltpu

IN_F = 4096
OUT_F = 4096
RANK = 64
TOPK = 8
SCALE = 2.0


def _fused_body(x_ref, at_ref, d_ref, bt_ref, o_ref):
    xb = x_ref[...]
    y = jnp.dot(xb.astype(jnp.bfloat16), at_ref[...].astype(jnp.bfloat16),
                preferred_element_type=jnp.float32)
    a = jnp.abs(y + d_ref[...])

    bt = a.shape[0]
    work = a.T
    iota = jax.lax.broadcasted_iota(jnp.int32, (RANK, bt), 0)
    keep = jnp.zeros((RANK, bt), jnp.float32)
    for _ in range(TOPK):
        m = jnp.max(work, axis=0, keepdims=True)
        first = jnp.min(jnp.where(work == m, iota, RANK), axis=0, keepdims=True)
        sel = iota == first
        keep = jnp.where(sel, 1.0, keep)
        work = jnp.where(sel, -1.0, work)
    masked_y = y * keep.T

    out = jnp.dot(masked_y.astype(jnp.bfloat16), bt_ref[...].astype(jnp.bfloat16),
                  preferred_element_type=jnp.float32)
    o_ref[...] = out * SCALE


def kernel(x, A, d, B):
    orig_shape = x.shape
    xt = x.reshape(-1, IN_F)
    n_tok = xt.shape[0]
    BT = 256
    grid = (n_tok // BT,)

    out = pl.pallas_call(
        _fused_body,
        grid=grid,
        in_specs=[
            pl.BlockSpec((BT, IN_F), lambda i: (i, 0)),
            pl.BlockSpec((IN_F, RANK), lambda i: (0, 0)),
            pl.BlockSpec((1, RANK), lambda i: (0, 0)),
            pl.BlockSpec((RANK, OUT_F), lambda i: (0, 0)),
        ],
        out_specs=pl.BlockSpec((BT, OUT_F), lambda i: (i, 0)),
        out_shape=jax.ShapeDtypeStruct((n_tok, OUT_F), jnp.float32),
        compiler_params=pltpu.CompilerParams(
            dimension_semantics=("parallel",)),
    )(xt, A.T, d.reshape(1, RANK), B.T)

    return out.reshape(orig_shape[:-1] + (OUT_F,))
